# baseline (speedup 1.0000x reference)
import functools
import math

import jax
import jax.numpy as jnp
from jax import lax
from jax.experimental import pallas as pl
from jax.experimental.pallas import tpu as pltpu
from jax.experimental.pallas import tpu_sc as plsc

EMB_D = 32
LANES = 16
NUM_CORES = 2
NUM_SUBCORES = 16
NUM_WORKERS = NUM_CORES * NUM_SUBCORES

B_DIM = 200
A_DIM = 4096
AB_PER = 4
ROWS = AB_PER * 128
UNITS = B_DIM * (A_DIM // 128) // AB_PER
UNITS_PER_W = UNITS // NUM_WORKERS
NBUF = 2
TRANS_WORDS = 4 * AB_PER * 1024
CH_WORDS = AB_PER * 1024


def _transpose_scale(rows_v, trans_v, scale, iota16):
    def line_step(l, _):
        c = ((l >> 5) << 3) | (l & 7)
        ab = (l >> 3) & 3
        colv = jnp.broadcast_to(c, (LANES,)).astype(jnp.int32)
        rbase = ab * 128
        obase = l * 128
        for blk in range(8):
            ridx = (rbase + blk * LANES) + iota16
            vec = plsc.load_gather(rows_v, [ridx, colv]) * scale
            trans_v[pl.ds(obase + blk * LANES, LANES)] = vec
        return 0

    lax.fori_loop(0, 128, line_step, 0)


def _body(table_hbm, idx_hbm, out_hbm,
          idx0, idx1, rows0, rows1, trans0, trans1,
          gsem0, gsem1, ssem0, ssem1, isem0, isem1):
    wid = lax.axis_index("s") * NUM_CORES + lax.axis_index("c")
    u0 = wid * UNITS_PER_W
    scale = jnp.float32(math.sqrt(EMB_D))
    iota16 = lax.iota(jnp.int32, LANES)

    slots = (
        (idx0, rows0, trans0, gsem0, ssem0, isem0),
        (idx1, rows1, trans1, gsem1, ssem1, isem1),
    )

    def out_word(u):
        b = u // 8
        q = u % 8
        return (b * 128 + q * AB_PER) * 1024

    for s in range(NBUF):
        idx_v, rows_v, _, gsem, _, _ = slots[s]
        pltpu.sync_copy(idx_hbm.at[pl.ds((u0 + s) * ROWS, ROWS)], idx_v)
        pltpu.async_copy(table_hbm.at[idx_v], rows_v, gsem)

    def outer(i, _):
        for s in range(NBUF):
            idx_v, rows_v, trans_v, gsem, ssem, isem = slots[s]
            u = u0 + i * NBUF + s
            nxt = u + NBUF
            last = u0 + UNITS_PER_W

            pltpu.make_async_copy(table_hbm.at[idx_v], rows_v, gsem).wait()

            @pl.when(nxt < last)
            def _():
                pltpu.async_copy(
                    idx_hbm.at[pl.ds(nxt * ROWS, ROWS)], idx_v, isem)

            @pl.when(u - NBUF >= u0)
            def _():
                pw = out_word(u - NBUF)
                for ch in range(4):
                    pltpu.make_async_copy(
                        trans_v.at[pl.ds(ch * CH_WORDS, CH_WORDS)],
                        out_hbm.at[pl.ds(pw + ch * 32 * 1024, CH_WORDS)],
                        ssem).wait()

            _transpose_scale(rows_v, trans_v, scale, iota16)

            w0 = out_word(u)
            for ch in range(4):
                pltpu.async_copy(
                    trans_v.at[pl.ds(ch * CH_WORDS, CH_WORDS)],
                    out_hbm.at[pl.ds(w0 + ch * 32 * 1024, CH_WORDS)],
                    ssem)

            @pl.when(nxt < last)
            def _():
                pltpu.make_async_copy(
                    idx_hbm.at[pl.ds(nxt * ROWS, ROWS)], idx_v, isem).wait()
                pltpu.async_copy(table_hbm.at[idx_v], rows_v, gsem)

        return 0

    lax.fori_loop(0, UNITS_PER_W // NBUF, outer, 0)

    for s in range(NBUF):
        _, _, trans_v, _, ssem, _ = slots[s]
        pw = out_word(u0 + UNITS_PER_W - NBUF + s)
        for ch in range(4):
            pltpu.make_async_copy(
                trans_v.at[pl.ds(ch * CH_WORDS, CH_WORDS)],
                out_hbm.at[pl.ds(pw + ch * 32 * 1024, CH_WORDS)],
                ssem).wait()


def _gather_transposed(table, idx_flat):
    mesh = plsc.VectorSubcoreMesh(core_axis_name="c", subcore_axis_name="s")
    k = functools.partial(
        pl.kernel,
        mesh=mesh,
        out_type=jax.ShapeDtypeStruct((A_DIM * B_DIM * EMB_D,), jnp.float32),
        compiler_params=pltpu.CompilerParams(
            use_tc_tiling_on_sc=False, needs_layout_passes=False,
            disable_bounds_checks=True),
        scratch_types=[
            pltpu.VMEM((ROWS,), jnp.int32),
            pltpu.VMEM((ROWS,), jnp.int32),
            pltpu.VMEM((ROWS, EMB_D), jnp.float32),
            pltpu.VMEM((ROWS, EMB_D), jnp.float32),
            pltpu.VMEM((TRANS_WORDS,), jnp.float32),
            pltpu.VMEM((TRANS_WORDS,), jnp.float32),
            pltpu.SemaphoreType.DMA,
            pltpu.SemaphoreType.DMA,
            pltpu.SemaphoreType.DMA,
            pltpu.SemaphoreType.DMA,
            pltpu.SemaphoreType.DMA,
            pltpu.SemaphoreType.DMA,
        ],
    )(_body)
    return k(table, idx_flat)


def kernel(tokens, table):
    idx_flat = tokens.T.reshape(A_DIM * B_DIM).astype(jnp.int32)
    flat = _gather_transposed(table, idx_flat)
    out5 = flat.reshape(B_DIM, 4, 32, 8, 128)
    return out5.transpose(2, 4, 0, 1, 3).reshape(A_DIM, B_DIM, EMB_D)


# speedup vs baseline: 1.1956x; 1.1956x over previous
import functools
import math

import jax
import jax.numpy as jnp
from jax import lax
from jax.experimental import pallas as pl
from jax.experimental.pallas import tpu as pltpu
from jax.experimental.pallas import tpu_sc as plsc

EMB_D = 32
LANES = 16
NUM_CORES = 2
NUM_SUBCORES = 16
NUM_WORKERS = NUM_CORES * NUM_SUBCORES

CHUNK = 800
NBUF = 2
ROWS_UNROLL = 4


def _scale_chunk(src_v, dst_v, scale):
    def scale_step(r, _):
        row = r * ROWS_UNROLL
        for u in range(ROWS_UNROLL):
            for j in range(EMB_D // LANES):
                sl = pl.ds(j * LANES, LANES)
                dst_v[row + u, sl] = src_v[row + u, sl] * scale
        return 0

    lax.fori_loop(0, CHUNK // ROWS_UNROLL, scale_step, 0)


def _body(table_hbm, idx_hbm, out_hbm,
          idx0, idx1, in0, in1, out0, out1,
          gsem0, gsem1, ssem0, ssem1, isem0, isem1):
    wid = lax.axis_index("s") * NUM_CORES + lax.axis_index("c")
    b_total = idx_hbm.shape[0]
    b_per_w = b_total // NUM_WORKERS
    n_chunks = b_per_w // CHUNK
    base = wid * b_per_w
    scale = jnp.float32(math.sqrt(EMB_D))

    slots = (
        (idx0, in0, out0, gsem0, ssem0, isem0),
        (idx1, in1, out1, gsem1, ssem1, isem1),
    )

    for b in range(NBUF):
        idx_v, in_v, _, gsem, _, _ = slots[b]
        off = base + b * CHUNK
        pltpu.sync_copy(idx_hbm.at[pl.ds(off, CHUNK)], idx_v)
        pltpu.async_copy(table_hbm.at[idx_v], in_v, gsem)

    def outer(i, _):
        for b in range(NBUF):
            idx_v, in_v, out_v, gsem, ssem, isem = slots[b]
            g = i * NBUF + b
            off = base + g * CHUNK
            nxt = g + NBUF

            pltpu.make_async_copy(table_hbm.at[idx_v], in_v, gsem).wait()

            @pl.when(nxt < n_chunks)
            def _():
                noff = base + nxt * CHUNK
                pltpu.async_copy(idx_hbm.at[pl.ds(noff, CHUNK)], idx_v, isem)

            @pl.when(g >= NBUF)
            def _():
                poff = base + (g - NBUF) * CHUNK
                pltpu.make_async_copy(
                    out_v, out_hbm.at[pl.ds(poff, CHUNK)], ssem).wait()

            _scale_chunk(in_v, out_v, scale)

            pltpu.async_copy(out_v, out_hbm.at[pl.ds(off, CHUNK)], ssem)

            @pl.when(nxt < n_chunks)
            def _():
                noff = base + nxt * CHUNK
                pltpu.make_async_copy(
                    idx_hbm.at[pl.ds(noff, CHUNK)], idx_v, isem).wait()
                pltpu.async_copy(table_hbm.at[idx_v], in_v, gsem)

        return 0

    lax.fori_loop(0, n_chunks // NBUF, outer, 0)

    for b in range(NBUF):
        _, _, out_v, _, ssem, _ = slots[b]
        off = base + (n_chunks - NBUF + b) * CHUNK
        pltpu.make_async_copy(out_v, out_hbm.at[pl.ds(off, CHUNK)], ssem).wait()


def _gather_scaled(table, idx):
    b_total = idx.shape[0]
    mesh = plsc.VectorSubcoreMesh(core_axis_name="c", subcore_axis_name="s")
    k = functools.partial(
        pl.kernel,
        mesh=mesh,
        out_type=jax.ShapeDtypeStruct((b_total, EMB_D), jnp.float32),
        compiler_params=pltpu.CompilerParams(use_tc_tiling_on_sc=False),
        scratch_types=[
            pltpu.VMEM((CHUNK,), jnp.int32),
            pltpu.VMEM((CHUNK,), jnp.int32),
            pltpu.VMEM((CHUNK, EMB_D), jnp.float32),
            pltpu.VMEM((CHUNK, EMB_D), jnp.float32),
            pltpu.VMEM((CHUNK, EMB_D), jnp.float32),
            pltpu.VMEM((CHUNK, EMB_D), jnp.float32),
            pltpu.SemaphoreType.DMA,
            pltpu.SemaphoreType.DMA,
            pltpu.SemaphoreType.DMA,
            pltpu.SemaphoreType.DMA,
            pltpu.SemaphoreType.DMA,
            pltpu.SemaphoreType.DMA,
        ],
    )(_body)
    return k(table, idx)


def kernel(tokens, table):
    b_total = tokens.size
    idx = tokens.reshape(b_total).astype(jnp.int32)
    out = _gather_scaled(table, idx)
    return out.reshape(*tokens.shape, EMB_D)


# speedup vs baseline: 1.8629x; 1.5582x over previous
import functools
import math

import jax
import jax.numpy as jnp
from jax import lax
from jax.experimental import pallas as pl
from jax.experimental.pallas import tpu as pltpu
from jax.experimental.pallas import tpu_sc as plsc

EMB_D = 32
LANES = 16
NUM_CORES = 2
NUM_SUBCORES = 16
NUM_WORKERS = NUM_CORES * NUM_SUBCORES

B_DIM = 200
A_DIM = 4096
AB_PER = 4
ROWS = AB_PER * 128
UNITS = B_DIM * (A_DIM // 128) // AB_PER
UNITS_PER_W = UNITS // NUM_WORKERS
NBUF = 2
TRANS_WORDS = 4 * AB_PER * 1024
CH_WORDS = AB_PER * 1024


def _rot(v, ridx):
    return v.at[ridx].get(mode="promise_in_bounds", unique_indices=True)


def _transpose16(regs, masks, ridxs):
    a = list(regs)
    for k in range(4):
        s = 1 << k
        m = masks[k]
        ridx = ridxs[k]
        for p in range(16):
            if p & s:
                continue
            q = p | s
            ap, aq = a[p], a[q]
            rp = _rot(ap, ridx)
            rq = _rot(aq, ridx)
            a[p] = jnp.where(m, ap, rq)
            a[q] = jnp.where(m, rp, aq)
    return a


def _transpose_scale(rows_v, trans_v, scale, masks, ridxs):
    def blk_step(t, _):
        ab = t >> 3
        alb = t & 7
        rbase = ab * 128 + alb * 16
        dyn = ab * 1024 + alb * 16
        for j in range(2):
            regs = [rows_v[rbase + i, pl.ds(16 * j, LANES)] for i in range(16)]
            tr = _transpose16(regs, masks, ridxs)
            for m in range(16):
                c = 16 * j + m
                off = (c >> 3) * 4096 + (c & 7) * 128
                trans_v[pl.ds(off + dyn, LANES)] = tr[m] * scale
        return 0

    lax.fori_loop(0, 32, blk_step, 0)


def _body(table_hbm, idx_hbm, out_hbm,
          idx0, idx1, rows0, rows1, trans0, trans1,
          gsem0, gsem1, ssem0, ssem1, isem0, isem1):
    wid = lax.axis_index("s") * NUM_CORES + lax.axis_index("c")
    u0 = wid * UNITS_PER_W
    scale = jnp.float32(math.sqrt(EMB_D))
    iota16 = lax.iota(jnp.int32, LANES)
    masks = tuple((iota16 & (1 << k)) == 0 for k in range(4))
    ridxs = tuple(iota16 ^ (1 << k) for k in range(4))

    slots = (
        (idx0, rows0, trans0, gsem0, ssem0, isem0),
        (idx1, rows1, trans1, gsem1, ssem1, isem1),
    )

    def out_word(u):
        b = u // 8
        q = u % 8
        return (b * 128 + q * AB_PER) * 1024

    for s in range(NBUF):
        idx_v, rows_v, _, gsem, _, _ = slots[s]
        pltpu.sync_copy(idx_hbm.at[pl.ds((u0 + s) * ROWS, ROWS)], idx_v)
        pltpu.async_copy(table_hbm.at[idx_v], rows_v, gsem)

    def outer(i, _):
        for s in range(NBUF):
            idx_v, rows_v, trans_v, gsem, ssem, isem = slots[s]
            u = u0 + i * NBUF + s
            nxt = u + NBUF
            last = u0 + UNITS_PER_W

            pltpu.make_async_copy(table_hbm.at[idx_v], rows_v, gsem).wait()

            @pl.when(nxt < last)
            def _():
                pltpu.async_copy(
                    idx_hbm.at[pl.ds(nxt * ROWS, ROWS)], idx_v, isem)

            @pl.when(u - NBUF >= u0)
            def _():
                pw = out_word(u - NBUF)
                for ch in range(4):
                    pltpu.make_async_copy(
                        trans_v.at[pl.ds(ch * CH_WORDS, CH_WORDS)],
                        out_hbm.at[pl.ds(pw + ch * 32 * 1024, CH_WORDS)],
                        ssem).wait()

            _transpose_scale(rows_v, trans_v, scale, masks, ridxs)

            w0 = out_word(u)
            for ch in range(4):
                pltpu.async_copy(
                    trans_v.at[pl.ds(ch * CH_WORDS, CH_WORDS)],
                    out_hbm.at[pl.ds(w0 + ch * 32 * 1024, CH_WORDS)],
                    ssem)

            @pl.when(nxt < last)
            def _():
                pltpu.make_async_copy(
                    idx_hbm.at[pl.ds(nxt * ROWS, ROWS)], idx_v, isem).wait()
                pltpu.async_copy(table_hbm.at[idx_v], rows_v, gsem)

        return 0

    lax.fori_loop(0, UNITS_PER_W // NBUF, outer, 0)

    for s in range(NBUF):
        _, _, trans_v, _, ssem, _ = slots[s]
        pw = out_word(u0 + UNITS_PER_W - NBUF + s)
        for ch in range(4):
            pltpu.make_async_copy(
                trans_v.at[pl.ds(ch * CH_WORDS, CH_WORDS)],
                out_hbm.at[pl.ds(pw + ch * 32 * 1024, CH_WORDS)],
                ssem).wait()


def _gather_transposed(table, idx_flat):
    mesh = plsc.VectorSubcoreMesh(core_axis_name="c", subcore_axis_name="s")
    k = functools.partial(
        pl.kernel,
        mesh=mesh,
        out_type=jax.ShapeDtypeStruct((A_DIM * B_DIM * EMB_D,), jnp.float32),
        compiler_params=pltpu.CompilerParams(
            use_tc_tiling_on_sc=False, needs_layout_passes=False),
        scratch_types=[
            pltpu.VMEM((ROWS,), jnp.int32),
            pltpu.VMEM((ROWS,), jnp.int32),
            pltpu.VMEM((ROWS, EMB_D), jnp.float32),
            pltpu.VMEM((ROWS, EMB_D), jnp.float32),
            pltpu.VMEM((TRANS_WORDS,), jnp.float32),
            pltpu.VMEM((TRANS_WORDS,), jnp.float32),
            pltpu.SemaphoreType.DMA,
            pltpu.SemaphoreType.DMA,
            pltpu.SemaphoreType.DMA,
            pltpu.SemaphoreType.DMA,
            pltpu.SemaphoreType.DMA,
            pltpu.SemaphoreType.DMA,
        ],
    )(_body)
    return k(table, idx_flat)


def kernel(tokens, table):
    idx_flat = tokens.T.reshape(A_DIM * B_DIM).astype(jnp.int32)
    flat = _gather_transposed(table, idx_flat)
    out5 = flat.reshape(B_DIM, 4, 32, 8, 128)
    return out5.transpose(2, 4, 0, 1, 3).reshape(A_DIM, B_DIM, EMB_D)
